# Initial kernel scaffold; baseline (speedup 1.0000x reference)
import jax
import jax.numpy as jnp
from jax.experimental import pallas as pl


def kernel(h, edge_index, W1, b1, W2, b2):
    raise NotImplementedError("write your pallas kernel here")


# speedup vs baseline: 3.0234x; 3.0234x over previous
import functools

import jax
import jax.numpy as jnp
from jax import lax
from jax.experimental import pallas as pl
from jax.experimental.pallas import tpu as pltpu
from jax.experimental.pallas import tpu_sc as plsc

N = 10000
D = 128
NPAD = 10240
E_PAD = 327680
CHUNK = 128
ROWS_PER_TILE = NPAD // 16
EDGES_PER_TILE = E_PAD // 32
CH_PER_TILE = EDGES_PER_TILE // CHUNK

_mesh = plsc.VectorSubcoreMesh(core_axis_name="c", subcore_axis_name="s")


@functools.partial(
    pl.kernel,
    out_type=jax.ShapeDtypeStruct((2, NPAD), jnp.float32),
    mesh=_mesh,
    scratch_types=[
        pltpu.VMEM((CHUNK,), jnp.int32),
        pltpu.VMEM((CHUNK,), jnp.float32),
        pltpu.VMEM_SHARED((NPAD,), jnp.float32),
    ],
)
def _deg(edges_hbm, zeros1_hbm, out_hbm, idx_v, ones_v, acc_sh):
    c = lax.axis_index("c")
    s = lax.axis_index("s")
    for j in range(CHUNK // 16):
        ones_v[pl.ds(j * 16, 16)] = jnp.ones((16,), jnp.float32)
    pltpu.sync_copy(zeros1_hbm, acc_sh.at[pl.ds(s * ROWS_PER_TILE, ROWS_PER_TILE)])
    plsc.subcore_barrier()
    base = s * (E_PAD // 16)

    def body(g, _):
        off = base + g * CHUNK
        pltpu.sync_copy(edges_hbm.at[c, pl.ds(off, CHUNK)], idx_v)
        pltpu.sync_copy(ones_v, acc_sh.at[idx_v], add=True)
        return _

    lax.fori_loop(0, E_PAD // 16 // CHUNK, body, 0)
    plsc.subcore_barrier()
    pltpu.sync_copy(
        acc_sh.at[pl.ds(s * ROWS_PER_TILE, ROWS_PER_TILE)],
        out_hbm.at[c, pl.ds(s * ROWS_PER_TILE, ROWS_PER_TILE)],
    )


@functools.partial(
    pl.kernel,
    out_type=jax.ShapeDtypeStruct((2, NPAD, D), jnp.float32),
    mesh=_mesh,
    scratch_types=[
        pltpu.VMEM((CHUNK,), jnp.int32),
        pltpu.VMEM((CHUNK,), jnp.int32),
        pltpu.VMEM((CHUNK, D), jnp.float32),
        pltpu.VMEM_SHARED((NPAD, D), jnp.float32),
        pltpu.SemaphoreType.DMA,
    ],
)
def _agg(y_hbm, src_hbm, dst_hbm, zeros2_hbm, out_hbm,
         src_v, dst_v, rows_v, acc_sh, sem):
    c = lax.axis_index("c")
    s = lax.axis_index("s")
    pltpu.sync_copy(zeros2_hbm, acc_sh.at[pl.ds(s * ROWS_PER_TILE, ROWS_PER_TILE)])
    plsc.subcore_barrier()

    base = (c * 16 + s) * EDGES_PER_TILE

    def body(g, _):
        off = base + g * CHUNK
        pltpu.sync_copy(src_hbm.at[pl.ds(off, CHUNK)], src_v)
        pltpu.sync_copy(dst_hbm.at[pl.ds(off, CHUNK)], dst_v)
        pltpu.async_copy(y_hbm.at[src_v], rows_v, sem).wait()
        pltpu.sync_copy(rows_v, acc_sh.at[dst_v], add=True)
        return _

    lax.fori_loop(0, CH_PER_TILE, body, 0)
    plsc.subcore_barrier()
    pltpu.sync_copy(
        acc_sh.at[pl.ds(s * ROWS_PER_TILE, ROWS_PER_TILE)],
        out_hbm.at[c, pl.ds(s * ROWS_PER_TILE, ROWS_PER_TILE)],
    )


_BLK = 1024
_GRID = NPAD // _BLK


def _dense1_body(h_ref, w_ref, dego_ref, y_ref):
    ns = lax.rsqrt(jnp.maximum(dego_ref[...], 1.0))
    y_ref[...] = jnp.dot(h_ref[...], w_ref[...],
                         preferred_element_type=jnp.float32) * ns


def _dense2_body(a_ref, degi_ref, dego_ref, b1_ref, w_ref, y_ref):
    nd = lax.rsqrt(jnp.maximum(degi_ref[...], 1.0))
    ns = lax.rsqrt(jnp.maximum(dego_ref[...], 1.0))
    x = (a_ref[0] + a_ref[1]) * nd + b1_ref[...]
    x = jnp.maximum(x, 0.0)
    y_ref[...] = jnp.dot(x, w_ref[...], preferred_element_type=jnp.float32) * ns


def _dense3_body(a_ref, degi_ref, b2_ref, o_ref):
    nd = lax.rsqrt(jnp.maximum(degi_ref[...], 1.0))
    z = (a_ref[0] + a_ref[1]) * nd + b2_ref[...]
    m = jnp.max(z, axis=1, keepdims=True)
    lse = jnp.log(jnp.sum(jnp.exp(z - m), axis=1, keepdims=True)) + m
    o_ref[...] = z - lse


def _dense1(h, W1, dego):
    return pl.pallas_call(
        _dense1_body,
        grid=(_GRID,),
        in_specs=[
            pl.BlockSpec((_BLK, D), lambda i: (i, 0)),
            pl.BlockSpec((D, D), lambda i: (0, 0)),
            pl.BlockSpec((_BLK, 1), lambda i: (i, 0)),
        ],
        out_specs=pl.BlockSpec((_BLK, D), lambda i: (i, 0)),
        out_shape=jax.ShapeDtypeStruct((NPAD, D), jnp.float32),
    )(h, W1, dego)


def _dense2(a, degi, dego, b1, W2):
    return pl.pallas_call(
        _dense2_body,
        grid=(_GRID,),
        in_specs=[
            pl.BlockSpec((2, _BLK, D), lambda i: (0, i, 0)),
            pl.BlockSpec((_BLK, 1), lambda i: (i, 0)),
            pl.BlockSpec((_BLK, 1), lambda i: (i, 0)),
            pl.BlockSpec((1, D), lambda i: (0, 0)),
            pl.BlockSpec((D, D), lambda i: (0, 0)),
        ],
        out_specs=pl.BlockSpec((_BLK, D), lambda i: (i, 0)),
        out_shape=jax.ShapeDtypeStruct((NPAD, D), jnp.float32),
    )(a, degi, dego, b1, W2)


def _dense3(a, degi, b2):
    return pl.pallas_call(
        _dense3_body,
        grid=(_GRID,),
        in_specs=[
            pl.BlockSpec((2, _BLK, D), lambda i: (0, i, 0)),
            pl.BlockSpec((_BLK, 1), lambda i: (i, 0)),
            pl.BlockSpec((1, D), lambda i: (0, 0)),
        ],
        out_specs=pl.BlockSpec((_BLK, D), lambda i: (i, 0)),
        out_shape=jax.ShapeDtypeStruct((NPAD, D), jnp.float32),
    )(a, degi, b2)


@jax.jit
def kernel(h, edge_index, W1, b1, W2, b2):
    src = edge_index[0]
    dst = edge_index[1]
    e = src.shape[0]
    pad = jnp.full((E_PAD - e,), N, jnp.int32)
    src_p = jnp.concatenate([src, pad])
    dst_p = jnp.concatenate([dst, pad])
    edges_p = jnp.stack([src_p, dst_p])
    h_p = jnp.pad(h, ((0, NPAD - N), (0, 0)))
    zeros1 = jnp.zeros((ROWS_PER_TILE,), jnp.float32)
    zeros2 = jnp.zeros((ROWS_PER_TILE, D), jnp.float32)

    degs = _deg(edges_p, zeros1)
    dego = degs[0][:, None]
    degi = degs[1][:, None]

    y1 = _dense1(h_p, W1, dego)
    p1 = _agg(y1, src_p, dst_p, zeros2)
    y2 = _dense2(p1, degi, dego, b1[None, :], W2)
    p2 = _agg(y2, src_p, dst_p, zeros2)
    out = _dense3(p2, degi, b2[None, :])
    return out[:N]
